# Initial kernel scaffold; baseline (speedup 1.0000x reference)
import jax
import jax.numpy as jnp
from jax.experimental import pallas as pl


def kernel(x, edge_index, W1, b1, W2, b2):
    raise NotImplementedError("write your pallas kernel here")


# speedup vs baseline: 19.6076x; 19.6076x over previous
import functools

import jax
import jax.numpy as jnp
from jax import lax
from jax.experimental import pallas as pl
from jax.experimental.pallas import tpu as pltpu
from jax.experimental.pallas import tpu_sc as plsc

N = 10000
E = 320000
D_IN = 128
D_HID = 16
D_OUT = 40
D_PAD = 48

NC, NS = 2, 16
NW = NC * NS
CH = 128
IDX_BLK = 8
NPAD = 10240
ROWS_PER_TILE = NPAD // NS
EPAD = NW * 80 * CH
CPW = EPAD // NW // CH
EROWS = EPAD // CH

BLK_R = 1024


_SC_MESH = plsc.VectorSubcoreMesh(core_axis_name="c", subcore_axis_name="s")
_SC_PARAMS = pltpu.CompilerParams(use_tc_tiling_on_sc=False)


@functools.partial(
    pl.kernel,
    out_type=jax.ShapeDtypeStruct((NC, NPAD), jnp.float32),
    mesh=_SC_MESH,
    compiler_params=_SC_PARAMS,
    scratch_types=[
        pltpu.VMEM((IDX_BLK, CH), jnp.int32),
        pltpu.VMEM((CH,), jnp.float32),
        pltpu.VMEM_SHARED((NPAD,), jnp.float32),
    ],
)
def _sc_degree(dst_hbm, zero_hbm, out_hbm, didx_v, ones_v, acc_sh):
    c = lax.axis_index("c")
    s = lax.axis_index("s")
    w = c * NS + s
    r0 = s * ROWS_PER_TILE
    for i in range(CH // 16):
        ones_v[pl.ds(i * 16, 16)] = jnp.ones((16,), jnp.float32)
    pltpu.sync_copy(zero_hbm.at[pl.ds(r0, ROWS_PER_TILE)],
                    acc_sh.at[pl.ds(r0, ROWS_PER_TILE)])
    plsc.subcore_barrier()

    def blk(i, carry):
        row0 = w * CPW + i * IDX_BLK
        pltpu.sync_copy(dst_hbm.at[pl.ds(row0, IDX_BLK)], didx_v)
        for j in range(IDX_BLK):
            pltpu.sync_copy(ones_v, acc_sh.at[didx_v.at[j]], add=True)
        return carry

    lax.fori_loop(0, CPW // IDX_BLK, blk, 0)
    plsc.subcore_barrier()
    pltpu.sync_copy(acc_sh.at[pl.ds(r0, ROWS_PER_TILE)],
                    out_hbm.at[c, pl.ds(r0, ROWS_PER_TILE)])


def _make_sc_agg(depth):
    @functools.partial(
        pl.kernel,
        out_type=jax.ShapeDtypeStruct((NC, NPAD, depth), jnp.float32),
        mesh=_SC_MESH,
        compiler_params=_SC_PARAMS,
        scratch_types=[
            pltpu.VMEM((IDX_BLK, CH), jnp.int32),
            pltpu.VMEM((IDX_BLK, CH), jnp.int32),
            pltpu.VMEM((CH, depth), jnp.float32),
            pltpu.VMEM_SHARED((NPAD, depth), jnp.float32),
        ],
    )
    def agg(g_hbm, src_hbm, dst_hbm, zero_hbm, out_hbm,
            sidx_v, didx_v, rows_v, acc_sh):
        c = lax.axis_index("c")
        s = lax.axis_index("s")
        w = c * NS + s
        r0 = s * ROWS_PER_TILE
        pltpu.sync_copy(zero_hbm.at[pl.ds(r0, ROWS_PER_TILE)],
                        acc_sh.at[pl.ds(r0, ROWS_PER_TILE)])
        plsc.subcore_barrier()

        def blk(i, carry):
            row0 = w * CPW + i * IDX_BLK
            pltpu.sync_copy(src_hbm.at[pl.ds(row0, IDX_BLK)], sidx_v)
            pltpu.sync_copy(dst_hbm.at[pl.ds(row0, IDX_BLK)], didx_v)
            for j in range(IDX_BLK):
                pltpu.sync_copy(g_hbm.at[sidx_v.at[j]], rows_v)
                pltpu.sync_copy(rows_v, acc_sh.at[didx_v.at[j]], add=True)
            return carry

        lax.fori_loop(0, CPW // IDX_BLK, blk, 0)
        plsc.subcore_barrier()
        pltpu.sync_copy(acc_sh.at[pl.ds(r0, ROWS_PER_TILE)],
                        out_hbm.at[c, pl.ds(r0, ROWS_PER_TILE)])

    return agg


_sc_agg_hid = _make_sc_agg(D_HID)
_sc_agg_out = _make_sc_agg(D_PAD)


def _tc1_body(x_ref, w_ref, cnt_ref, g_ref, dinv_ref):
    i = pl.program_id(0)
    h = jnp.dot(x_ref[...], w_ref[...], preferred_element_type=jnp.float32)
    csum = cnt_ref[:, 0:1] + cnt_ref[:, 1:2]
    rows = jax.lax.broadcasted_iota(jnp.int32, (BLK_R, 1), 0) + i * BLK_R
    dinv = jnp.where(rows < N, jax.lax.rsqrt(csum + 1.0), 0.0)
    dinv_ref[...] = dinv
    g_ref[...] = h * dinv


_tc1 = pl.pallas_call(
    _tc1_body,
    grid=(NPAD // BLK_R,),
    in_specs=[
        pl.BlockSpec((BLK_R, D_IN), lambda i: (i, 0)),
        pl.BlockSpec((D_IN, D_HID), lambda i: (0, 0)),
        pl.BlockSpec((BLK_R, 2), lambda i: (i, 0)),
    ],
    out_specs=[
        pl.BlockSpec((BLK_R, D_HID), lambda i: (i, 0)),
        pl.BlockSpec((BLK_R, 1), lambda i: (i, 0)),
    ],
    out_shape=[
        jax.ShapeDtypeStruct((NPAD, D_HID), jnp.float32),
        jax.ShapeDtypeStruct((NPAD, 1), jnp.float32),
    ],
)


def _tc2_body(a0_ref, a1_ref, g1_ref, dinv_ref, b1_ref, w2_ref, g2_ref):
    agg = a0_ref[...] + a1_ref[...] + g1_ref[...]
    dinv = dinv_ref[...]
    h1 = jnp.maximum(agg * dinv + b1_ref[...], 0.0)
    g2_ref[...] = jnp.dot(h1, w2_ref[...],
                          preferred_element_type=jnp.float32) * dinv


_tc2 = pl.pallas_call(
    _tc2_body,
    grid=(NPAD // BLK_R,),
    in_specs=[
        pl.BlockSpec((BLK_R, D_HID), lambda i: (i, 0)),
        pl.BlockSpec((BLK_R, D_HID), lambda i: (i, 0)),
        pl.BlockSpec((BLK_R, D_HID), lambda i: (i, 0)),
        pl.BlockSpec((BLK_R, 1), lambda i: (i, 0)),
        pl.BlockSpec((1, D_HID), lambda i: (0, 0)),
        pl.BlockSpec((D_HID, D_PAD), lambda i: (0, 0)),
    ],
    out_specs=pl.BlockSpec((BLK_R, D_PAD), lambda i: (i, 0)),
    out_shape=jax.ShapeDtypeStruct((NPAD, D_PAD), jnp.float32),
)


def _tc3_body(a0_ref, a1_ref, g2_ref, dinv_ref, b2_ref, o_ref):
    o = (a0_ref[...] + a1_ref[...] + g2_ref[...]) * dinv_ref[...] + b2_ref[...]
    col = jax.lax.broadcasted_iota(jnp.int32, (1, D_PAD), 1)
    o = jnp.where(col < D_OUT, o, -1e30)
    m = jnp.max(o, axis=1, keepdims=True)
    lse = m + jnp.log(jnp.sum(jnp.exp(o - m), axis=1, keepdims=True))
    o_ref[...] = o - lse


_tc3 = pl.pallas_call(
    _tc3_body,
    grid=(NPAD // BLK_R,),
    in_specs=[
        pl.BlockSpec((BLK_R, D_PAD), lambda i: (i, 0)),
        pl.BlockSpec((BLK_R, D_PAD), lambda i: (i, 0)),
        pl.BlockSpec((BLK_R, D_PAD), lambda i: (i, 0)),
        pl.BlockSpec((BLK_R, 1), lambda i: (i, 0)),
        pl.BlockSpec((1, D_PAD), lambda i: (0, 0)),
    ],
    out_specs=pl.BlockSpec((BLK_R, D_PAD), lambda i: (i, 0)),
    out_shape=jax.ShapeDtypeStruct((NPAD, D_PAD), jnp.float32),
)


def kernel(x, edge_index, W1, b1, W2, b2):
    pad_idx = jnp.full((EPAD - E,), NPAD - 1, dtype=jnp.int32)
    src2 = jnp.concatenate([edge_index[0], pad_idx]).reshape(EROWS, CH)
    dst2 = jnp.concatenate([edge_index[1], pad_idx]).reshape(EROWS, CH)

    cnt = _sc_degree(dst2, jnp.zeros((NPAD,), jnp.float32))
    xpad = jnp.concatenate(
        [x, jnp.zeros((NPAD - N, D_IN), jnp.float32)], axis=0)
    g1, dinv = _tc1(xpad, W1, cnt.T)

    agg1 = _sc_agg_hid(g1, src2, dst2, jnp.zeros((NPAD, D_HID), jnp.float32))
    W2p = jnp.concatenate(
        [W2, jnp.zeros((D_HID, D_PAD - D_OUT), jnp.float32)], axis=1)
    b2p = jnp.concatenate(
        [b2, jnp.zeros((D_PAD - D_OUT,), jnp.float32)]).reshape(1, D_PAD)
    g2 = _tc2(agg1[0], agg1[1], g1, dinv, b1.reshape(1, D_HID), W2p)

    agg2 = _sc_agg_out(g2, src2, dst2, jnp.zeros((NPAD, D_PAD), jnp.float32))
    o = _tc3(agg2[0], agg2[1], g2, dinv, b2p)
    return o[:N, :D_OUT]


# speedup vs baseline: 20.7678x; 1.0592x over previous
import functools

import jax
import jax.numpy as jnp
from jax import lax
from jax.experimental import pallas as pl
from jax.experimental.pallas import tpu as pltpu
from jax.experimental.pallas import tpu_sc as plsc

N = 10000
E = 320000
D_IN = 128
D_HID = 16
D_OUT = 40
D_PAD = 48

NC, NS = 2, 16
NW = NC * NS
CH = 128
IDX_BLK = 8
NPAD = 10240
ROWS_PER_TILE = NPAD // NS
EPAD = NW * 80 * CH
CPW = EPAD // NW // CH
EROWS = EPAD // CH

BLK_R = 1024


_SC_MESH = plsc.VectorSubcoreMesh(core_axis_name="c", subcore_axis_name="s")
_SC_PARAMS = pltpu.CompilerParams(use_tc_tiling_on_sc=False)


@functools.partial(
    pl.kernel,
    out_type=jax.ShapeDtypeStruct((NC, NPAD), jnp.float32),
    mesh=_SC_MESH,
    compiler_params=_SC_PARAMS,
    scratch_types=[
        pltpu.VMEM((IDX_BLK, CH), jnp.int32),
        pltpu.VMEM((CH,), jnp.float32),
        pltpu.VMEM_SHARED((NPAD,), jnp.float32),
    ],
)
def _sc_degree(dst_hbm, zero_hbm, out_hbm, didx_v, ones_v, acc_sh):
    c = lax.axis_index("c")
    s = lax.axis_index("s")
    w = c * NS + s
    r0 = s * ROWS_PER_TILE
    for i in range(CH // 16):
        ones_v[pl.ds(i * 16, 16)] = jnp.ones((16,), jnp.float32)
    pltpu.sync_copy(zero_hbm.at[pl.ds(r0, ROWS_PER_TILE)],
                    acc_sh.at[pl.ds(r0, ROWS_PER_TILE)])
    plsc.subcore_barrier()

    def blk(i, carry):
        row0 = w * CPW + i * IDX_BLK
        pltpu.sync_copy(dst_hbm.at[pl.ds(row0, IDX_BLK)], didx_v)
        for j in range(IDX_BLK):
            pltpu.sync_copy(ones_v, acc_sh.at[didx_v.at[j]], add=True)
        return carry

    lax.fori_loop(0, CPW // IDX_BLK, blk, 0)
    plsc.subcore_barrier()
    pltpu.sync_copy(acc_sh.at[pl.ds(r0, ROWS_PER_TILE)],
                    out_hbm.at[c, pl.ds(r0, ROWS_PER_TILE)])


def _make_sc_agg(depth):
    @functools.partial(
        pl.kernel,
        out_type=jax.ShapeDtypeStruct((NC, NPAD, depth), jnp.float32),
        mesh=_SC_MESH,
        compiler_params=_SC_PARAMS,
        scratch_types=[
            pltpu.VMEM((IDX_BLK, CH), jnp.int32),
            pltpu.VMEM((IDX_BLK, CH), jnp.int32),
            pltpu.VMEM((CH, depth), jnp.float32),
            pltpu.VMEM((CH, depth), jnp.float32),
            pltpu.VMEM_SHARED((NPAD, depth), jnp.float32),
            pltpu.SemaphoreType.DMA,
            pltpu.SemaphoreType.DMA,
        ],
    )
    def agg(g_hbm, src_hbm, dst_hbm, zero_hbm, out_hbm,
            sidx_v, didx_v, rows0_v, rows1_v, acc_sh, sem0, sem1):
        c = lax.axis_index("c")
        s = lax.axis_index("s")
        w = c * NS + s
        r0 = s * ROWS_PER_TILE
        rows = (rows0_v, rows1_v)
        sems = (sem0, sem1)
        pltpu.sync_copy(zero_hbm.at[pl.ds(r0, ROWS_PER_TILE)],
                        acc_sh.at[pl.ds(r0, ROWS_PER_TILE)])
        plsc.subcore_barrier()

        def blk(i, carry):
            row0 = w * CPW + i * IDX_BLK
            pltpu.sync_copy(src_hbm.at[pl.ds(row0, IDX_BLK)], sidx_v)
            pltpu.sync_copy(dst_hbm.at[pl.ds(row0, IDX_BLK)], didx_v)
            cps = [None, None]
            cps[0] = pltpu.async_copy(
                g_hbm.at[sidx_v.at[0]], rows[0], sems[0])
            for j in range(IDX_BLK):
                cps[j % 2].wait()
                if j + 1 < IDX_BLK:
                    cps[(j + 1) % 2] = pltpu.async_copy(
                        g_hbm.at[sidx_v.at[j + 1]],
                        rows[(j + 1) % 2], sems[(j + 1) % 2])
                pltpu.sync_copy(rows[j % 2], acc_sh.at[didx_v.at[j]],
                                add=True)
            return carry

        lax.fori_loop(0, CPW // IDX_BLK, blk, 0)
        plsc.subcore_barrier()
        pltpu.sync_copy(acc_sh.at[pl.ds(r0, ROWS_PER_TILE)],
                        out_hbm.at[c, pl.ds(r0, ROWS_PER_TILE)])

    return agg


_sc_agg_hid = _make_sc_agg(D_HID)
_sc_agg_out = _make_sc_agg(D_PAD)


def _tc1_body(x_ref, w_ref, cnt_ref, g_ref, dinv_ref):
    i = pl.program_id(0)
    h = jnp.dot(x_ref[...], w_ref[...], preferred_element_type=jnp.float32)
    csum = cnt_ref[:, 0:1] + cnt_ref[:, 1:2]
    rows = jax.lax.broadcasted_iota(jnp.int32, (BLK_R, 1), 0) + i * BLK_R
    dinv = jnp.where(rows < N, jax.lax.rsqrt(csum + 1.0), 0.0)
    dinv_ref[...] = dinv
    g_ref[...] = h * dinv


_tc1 = pl.pallas_call(
    _tc1_body,
    grid=(NPAD // BLK_R,),
    in_specs=[
        pl.BlockSpec((BLK_R, D_IN), lambda i: (i, 0)),
        pl.BlockSpec((D_IN, D_HID), lambda i: (0, 0)),
        pl.BlockSpec((BLK_R, 2), lambda i: (i, 0)),
    ],
    out_specs=[
        pl.BlockSpec((BLK_R, D_HID), lambda i: (i, 0)),
        pl.BlockSpec((BLK_R, 1), lambda i: (i, 0)),
    ],
    out_shape=[
        jax.ShapeDtypeStruct((NPAD, D_HID), jnp.float32),
        jax.ShapeDtypeStruct((NPAD, 1), jnp.float32),
    ],
)


def _tc2_body(a0_ref, a1_ref, g1_ref, dinv_ref, b1_ref, w2_ref, g2_ref):
    agg = a0_ref[...] + a1_ref[...] + g1_ref[...]
    dinv = dinv_ref[...]
    h1 = jnp.maximum(agg * dinv + b1_ref[...], 0.0)
    g2_ref[...] = jnp.dot(h1, w2_ref[...],
                          preferred_element_type=jnp.float32) * dinv


_tc2 = pl.pallas_call(
    _tc2_body,
    grid=(NPAD // BLK_R,),
    in_specs=[
        pl.BlockSpec((BLK_R, D_HID), lambda i: (i, 0)),
        pl.BlockSpec((BLK_R, D_HID), lambda i: (i, 0)),
        pl.BlockSpec((BLK_R, D_HID), lambda i: (i, 0)),
        pl.BlockSpec((BLK_R, 1), lambda i: (i, 0)),
        pl.BlockSpec((1, D_HID), lambda i: (0, 0)),
        pl.BlockSpec((D_HID, D_PAD), lambda i: (0, 0)),
    ],
    out_specs=pl.BlockSpec((BLK_R, D_PAD), lambda i: (i, 0)),
    out_shape=jax.ShapeDtypeStruct((NPAD, D_PAD), jnp.float32),
)


def _tc3_body(a0_ref, a1_ref, g2_ref, dinv_ref, b2_ref, o_ref):
    o = (a0_ref[...] + a1_ref[...] + g2_ref[...]) * dinv_ref[...] + b2_ref[...]
    col = jax.lax.broadcasted_iota(jnp.int32, (1, D_PAD), 1)
    o = jnp.where(col < D_OUT, o, -1e30)
    m = jnp.max(o, axis=1, keepdims=True)
    lse = m + jnp.log(jnp.sum(jnp.exp(o - m), axis=1, keepdims=True))
    o_ref[...] = o - lse


_tc3 = pl.pallas_call(
    _tc3_body,
    grid=(NPAD // BLK_R,),
    in_specs=[
        pl.BlockSpec((BLK_R, D_PAD), lambda i: (i, 0)),
        pl.BlockSpec((BLK_R, D_PAD), lambda i: (i, 0)),
        pl.BlockSpec((BLK_R, D_PAD), lambda i: (i, 0)),
        pl.BlockSpec((BLK_R, 1), lambda i: (i, 0)),
        pl.BlockSpec((1, D_PAD), lambda i: (0, 0)),
    ],
    out_specs=pl.BlockSpec((BLK_R, D_PAD), lambda i: (i, 0)),
    out_shape=jax.ShapeDtypeStruct((NPAD, D_PAD), jnp.float32),
)


def kernel(x, edge_index, W1, b1, W2, b2):
    pad_idx = jnp.full((EPAD - E,), NPAD - 1, dtype=jnp.int32)
    src2 = jnp.concatenate([edge_index[0], pad_idx]).reshape(EROWS, CH)
    dst2 = jnp.concatenate([edge_index[1], pad_idx]).reshape(EROWS, CH)

    cnt = _sc_degree(dst2, jnp.zeros((NPAD,), jnp.float32))
    xpad = jnp.concatenate(
        [x, jnp.zeros((NPAD - N, D_IN), jnp.float32)], axis=0)
    g1, dinv = _tc1(xpad, W1, cnt.T)

    agg1 = _sc_agg_hid(g1, src2, dst2, jnp.zeros((NPAD, D_HID), jnp.float32))
    W2p = jnp.concatenate(
        [W2, jnp.zeros((D_HID, D_PAD - D_OUT), jnp.float32)], axis=1)
    b2p = jnp.concatenate(
        [b2, jnp.zeros((D_PAD - D_OUT,), jnp.float32)]).reshape(1, D_PAD)
    g2 = _tc2(agg1[0], agg1[1], g1, dinv, b1.reshape(1, D_HID), W2p)

    agg2 = _sc_agg_out(g2, src2, dst2, jnp.zeros((NPAD, D_PAD), jnp.float32))
    o = _tc3(agg2[0], agg2[1], g2, dinv, b2p)
    return o[:N, :D_OUT]


# speedup vs baseline: 21.8604x; 1.0526x over previous
import functools

import jax
import jax.numpy as jnp
from jax import lax
from jax.experimental import pallas as pl
from jax.experimental.pallas import tpu as pltpu
from jax.experimental.pallas import tpu_sc as plsc

N = 10000
E = 320000
D_IN = 128
D_HID = 16
D_OUT = 40
D_PAD = 40

NC, NS = 2, 16
NW = NC * NS
CH = 128
IDX_BLK = 8
NPAD = 10240
ROWS_PER_TILE = NPAD // NS
EPAD = NW * 80 * CH
CPW = EPAD // NW // CH
EROWS = EPAD // CH

BLK_R = 1024


_SC_MESH = plsc.VectorSubcoreMesh(core_axis_name="c", subcore_axis_name="s")
_SC_PARAMS = pltpu.CompilerParams(use_tc_tiling_on_sc=False)


@functools.partial(
    pl.kernel,
    out_type=jax.ShapeDtypeStruct((NC, NPAD), jnp.float32),
    mesh=_SC_MESH,
    compiler_params=_SC_PARAMS,
    scratch_types=[
        pltpu.VMEM((IDX_BLK, CH), jnp.int32),
        pltpu.VMEM((CH,), jnp.float32),
        pltpu.VMEM_SHARED((NPAD,), jnp.float32),
    ],
)
def _sc_degree(dst_hbm, zero_hbm, out_hbm, didx_v, ones_v, acc_sh):
    c = lax.axis_index("c")
    s = lax.axis_index("s")
    w = c * NS + s
    r0 = s * ROWS_PER_TILE
    for i in range(CH // 16):
        ones_v[pl.ds(i * 16, 16)] = jnp.ones((16,), jnp.float32)
    pltpu.sync_copy(zero_hbm.at[pl.ds(r0, ROWS_PER_TILE)],
                    acc_sh.at[pl.ds(r0, ROWS_PER_TILE)])
    plsc.subcore_barrier()

    def blk(i, carry):
        row0 = w * CPW + i * IDX_BLK
        pltpu.sync_copy(dst_hbm.at[pl.ds(row0, IDX_BLK)], didx_v)
        for j in range(IDX_BLK):
            pltpu.sync_copy(ones_v, acc_sh.at[didx_v.at[j]], add=True)
        return carry

    lax.fori_loop(0, CPW // IDX_BLK, blk, 0)
    plsc.subcore_barrier()
    pltpu.sync_copy(acc_sh.at[pl.ds(r0, ROWS_PER_TILE)],
                    out_hbm.at[c, pl.ds(r0, ROWS_PER_TILE)])


def _make_sc_agg(depth):
    @functools.partial(
        pl.kernel,
        out_type=jax.ShapeDtypeStruct((NC, NPAD, depth), jnp.float32),
        mesh=_SC_MESH,
        compiler_params=_SC_PARAMS,
        scratch_types=[
            pltpu.VMEM((IDX_BLK, CH), jnp.int32),
            pltpu.VMEM((IDX_BLK, CH), jnp.int32),
            pltpu.VMEM((CH, depth), jnp.float32),
            pltpu.VMEM((CH, depth), jnp.float32),
            pltpu.VMEM_SHARED((NPAD, depth), jnp.float32),
            pltpu.SemaphoreType.DMA,
            pltpu.SemaphoreType.DMA,
        ],
    )
    def agg(g_hbm, src_hbm, dst_hbm, zero_hbm, out_hbm,
            sidx_v, didx_v, rows0_v, rows1_v, acc_sh, sem0, sem1):
        c = lax.axis_index("c")
        s = lax.axis_index("s")
        w = c * NS + s
        r0 = s * ROWS_PER_TILE
        rows = (rows0_v, rows1_v)
        sems = (sem0, sem1)
        pltpu.sync_copy(zero_hbm.at[pl.ds(r0, ROWS_PER_TILE)],
                        acc_sh.at[pl.ds(r0, ROWS_PER_TILE)])
        plsc.subcore_barrier()

        def blk(i, carry):
            row0 = w * CPW + i * IDX_BLK
            pltpu.sync_copy(src_hbm.at[pl.ds(row0, IDX_BLK)], sidx_v)
            pltpu.sync_copy(dst_hbm.at[pl.ds(row0, IDX_BLK)], didx_v)
            cps = [None, None]
            cps[0] = pltpu.async_copy(
                g_hbm.at[sidx_v.at[0]], rows[0], sems[0])
            for j in range(IDX_BLK):
                cps[j % 2].wait()
                if j + 1 < IDX_BLK:
                    cps[(j + 1) % 2] = pltpu.async_copy(
                        g_hbm.at[sidx_v.at[j + 1]],
                        rows[(j + 1) % 2], sems[(j + 1) % 2])
                pltpu.sync_copy(rows[j % 2], acc_sh.at[didx_v.at[j]],
                                add=True)
            return carry

        lax.fori_loop(0, CPW // IDX_BLK, blk, 0)
        plsc.subcore_barrier()
        pltpu.sync_copy(acc_sh.at[pl.ds(r0, ROWS_PER_TILE)],
                        out_hbm.at[c, pl.ds(r0, ROWS_PER_TILE)])

    return agg


_sc_agg_hid = _make_sc_agg(D_HID)
_sc_agg_out = _make_sc_agg(D_PAD)


def _tc1_body(x_ref, w_ref, cnt_ref, g_ref, dinv_ref):
    i = pl.program_id(0)
    h = jnp.dot(x_ref[...], w_ref[...], preferred_element_type=jnp.float32)
    csum = cnt_ref[:, 0:1] + cnt_ref[:, 1:2]
    rows = jax.lax.broadcasted_iota(jnp.int32, (BLK_R, 1), 0) + i * BLK_R
    dinv = jnp.where(rows < N, jax.lax.rsqrt(csum + 1.0), 0.0)
    dinv_ref[...] = dinv
    g_ref[...] = h * dinv


_tc1 = pl.pallas_call(
    _tc1_body,
    grid=(NPAD // BLK_R,),
    in_specs=[
        pl.BlockSpec((BLK_R, D_IN), lambda i: (i, 0)),
        pl.BlockSpec((D_IN, D_HID), lambda i: (0, 0)),
        pl.BlockSpec((BLK_R, 2), lambda i: (i, 0)),
    ],
    out_specs=[
        pl.BlockSpec((BLK_R, D_HID), lambda i: (i, 0)),
        pl.BlockSpec((BLK_R, 1), lambda i: (i, 0)),
    ],
    out_shape=[
        jax.ShapeDtypeStruct((NPAD, D_HID), jnp.float32),
        jax.ShapeDtypeStruct((NPAD, 1), jnp.float32),
    ],
)


def _tc2_body(a0_ref, a1_ref, g1_ref, dinv_ref, b1_ref, w2_ref, g2_ref):
    agg = a0_ref[...] + a1_ref[...] + g1_ref[...]
    dinv = dinv_ref[...]
    h1 = jnp.maximum(agg * dinv + b1_ref[...], 0.0)
    g2_ref[...] = jnp.dot(h1, w2_ref[...],
                          preferred_element_type=jnp.float32) * dinv


_tc2 = pl.pallas_call(
    _tc2_body,
    grid=(NPAD // BLK_R,),
    in_specs=[
        pl.BlockSpec((BLK_R, D_HID), lambda i: (i, 0)),
        pl.BlockSpec((BLK_R, D_HID), lambda i: (i, 0)),
        pl.BlockSpec((BLK_R, D_HID), lambda i: (i, 0)),
        pl.BlockSpec((BLK_R, 1), lambda i: (i, 0)),
        pl.BlockSpec((1, D_HID), lambda i: (0, 0)),
        pl.BlockSpec((D_HID, D_PAD), lambda i: (0, 0)),
    ],
    out_specs=pl.BlockSpec((BLK_R, D_PAD), lambda i: (i, 0)),
    out_shape=jax.ShapeDtypeStruct((NPAD, D_PAD), jnp.float32),
)


def _tc3_body(a0_ref, a1_ref, g2_ref, dinv_ref, b2_ref, o_ref):
    o = (a0_ref[...] + a1_ref[...] + g2_ref[...]) * dinv_ref[...] + b2_ref[...]
    m = jnp.max(o, axis=1, keepdims=True)
    lse = m + jnp.log(jnp.sum(jnp.exp(o - m), axis=1, keepdims=True))
    o_ref[...] = o - lse


_tc3 = pl.pallas_call(
    _tc3_body,
    grid=(NPAD // BLK_R,),
    in_specs=[
        pl.BlockSpec((BLK_R, D_PAD), lambda i: (i, 0)),
        pl.BlockSpec((BLK_R, D_PAD), lambda i: (i, 0)),
        pl.BlockSpec((BLK_R, D_PAD), lambda i: (i, 0)),
        pl.BlockSpec((BLK_R, 1), lambda i: (i, 0)),
        pl.BlockSpec((1, D_PAD), lambda i: (0, 0)),
    ],
    out_specs=pl.BlockSpec((BLK_R, D_PAD), lambda i: (i, 0)),
    out_shape=jax.ShapeDtypeStruct((NPAD, D_PAD), jnp.float32),
)


def kernel(x, edge_index, W1, b1, W2, b2):
    pad_idx = jnp.full((EPAD - E,), NPAD - 1, dtype=jnp.int32)
    src2 = jnp.concatenate([edge_index[0], pad_idx]).reshape(EROWS, CH)
    dst2 = jnp.concatenate([edge_index[1], pad_idx]).reshape(EROWS, CH)

    cnt = _sc_degree(dst2, jnp.zeros((NPAD,), jnp.float32))
    xpad = jnp.concatenate(
        [x, jnp.zeros((NPAD - N, D_IN), jnp.float32)], axis=0)
    g1, dinv = _tc1(xpad, W1, cnt.T)

    agg1 = _sc_agg_hid(g1, src2, dst2, jnp.zeros((NPAD, D_HID), jnp.float32))
    g2 = _tc2(agg1[0], agg1[1], g1, dinv, b1.reshape(1, D_HID), W2)

    agg2 = _sc_agg_out(g2, src2, dst2, jnp.zeros((NPAD, D_PAD), jnp.float32))
    o = _tc3(agg2[0], agg2[1], g2, dinv, b2.reshape(1, D_PAD))
    return o[:N]


# speedup vs baseline: 24.6703x; 1.1285x over previous
import functools

import jax
import jax.numpy as jnp
from jax import lax
from jax.experimental import pallas as pl
from jax.experimental.pallas import tpu as pltpu
from jax.experimental.pallas import tpu_sc as plsc

N = 10000
E = 320000
D_IN = 128
D_HID = 16
D_OUT = 40
D_PAD = 40

NC, NS = 2, 16
NW = NC * NS
CH = 128
IDX_BLK = 8
NPAD = 10240
ROWS_PER_TILE = NPAD // NS
EPAD = NW * 80 * CH
CPW = EPAD // NW // CH
EROWS = EPAD // CH

BLK_R = 1024


_SC_MESH = plsc.VectorSubcoreMesh(core_axis_name="c", subcore_axis_name="s")
_SC_PARAMS = pltpu.CompilerParams(use_tc_tiling_on_sc=False)


@functools.partial(
    pl.kernel,
    out_type=jax.ShapeDtypeStruct((NC, NPAD), jnp.float32),
    mesh=_SC_MESH,
    compiler_params=_SC_PARAMS,
    scratch_types=[
        pltpu.VMEM((IDX_BLK, CH), jnp.int32),
        pltpu.VMEM((CH,), jnp.float32),
        pltpu.VMEM_SHARED((NPAD,), jnp.float32),
    ],
)
def _sc_degree(dst_hbm, zero_hbm, out_hbm, didx_v, ones_v, acc_sh):
    c = lax.axis_index("c")
    s = lax.axis_index("s")
    w = c * NS + s
    r0 = s * ROWS_PER_TILE
    for i in range(CH // 16):
        ones_v[pl.ds(i * 16, 16)] = jnp.ones((16,), jnp.float32)
    pltpu.sync_copy(zero_hbm.at[pl.ds(r0, ROWS_PER_TILE)],
                    acc_sh.at[pl.ds(r0, ROWS_PER_TILE)])
    plsc.subcore_barrier()

    def blk(i, carry):
        row0 = w * CPW + i * IDX_BLK
        pltpu.sync_copy(dst_hbm.at[pl.ds(row0, IDX_BLK)], didx_v)
        for j in range(IDX_BLK):
            pltpu.sync_copy(ones_v, acc_sh.at[didx_v.at[j]], add=True)
        return carry

    lax.fori_loop(0, CPW // IDX_BLK, blk, 0)
    plsc.subcore_barrier()
    pltpu.sync_copy(acc_sh.at[pl.ds(r0, ROWS_PER_TILE)],
                    out_hbm.at[c, pl.ds(r0, ROWS_PER_TILE)])


def _make_sc_agg(depth):
    @functools.partial(
        pl.kernel,
        out_type=jax.ShapeDtypeStruct((NC, NPAD, depth), jnp.float32),
        mesh=_SC_MESH,
        compiler_params=_SC_PARAMS,
        scratch_types=(
            [pltpu.VMEM((IDX_BLK, CH), jnp.int32),
             pltpu.VMEM((IDX_BLK, CH), jnp.int32)]
            + [pltpu.VMEM((CH, depth), jnp.float32) for _ in range(IDX_BLK)]
            + [pltpu.VMEM_SHARED((NPAD, depth), jnp.float32)]
            + [pltpu.SemaphoreType.DMA for _ in range(IDX_BLK + 2)]
        ),
    )
    def agg(g_hbm, src_hbm, dst_hbm, zero_hbm, out_hbm,
            sidx_v, didx_v, *scr):
        rows = scr[:IDX_BLK]
        acc_sh = scr[IDX_BLK]
        gsem = scr[IDX_BLK + 1:2 * IDX_BLK + 1]
        ssem = scr[2 * IDX_BLK + 1:]
        c = lax.axis_index("c")
        s = lax.axis_index("s")
        w = c * NS + s
        r0 = s * ROWS_PER_TILE
        grp = IDX_BLK // 2
        pltpu.sync_copy(zero_hbm.at[pl.ds(r0, ROWS_PER_TILE)],
                        acc_sh.at[pl.ds(r0, ROWS_PER_TILE)])
        plsc.subcore_barrier()

        def drain_scat(g):
            for _ in range(grp):
                pltpu.make_async_copy(
                    g_hbm.at[pl.ds(0, CH)], rows[0], ssem[g]).wait()

        def blk(i, carry):
            row0 = w * CPW + i * IDX_BLK
            pltpu.sync_copy(src_hbm.at[pl.ds(row0, IDX_BLK)], sidx_v)
            pltpu.sync_copy(dst_hbm.at[pl.ds(row0, IDX_BLK)], didx_v)
            for g in range(2):
                @pl.when(i >= 1)
                def _(g=g):
                    drain_scat(g)
                cps = []
                for j in range(grp):
                    k = g * grp + j
                    cps.append(pltpu.async_copy(
                        g_hbm.at[sidx_v.at[k]], rows[k], gsem[k]))
                for j in range(grp):
                    k = g * grp + j
                    cps[j].wait()
                    pltpu.async_copy(rows[k], acc_sh.at[didx_v.at[k]],
                                     ssem[g], add=True)
            return carry

        lax.fori_loop(0, CPW // IDX_BLK, blk, 0)
        for g in range(2):
            drain_scat(g)
        plsc.subcore_barrier()
        pltpu.sync_copy(acc_sh.at[pl.ds(r0, ROWS_PER_TILE)],
                        out_hbm.at[c, pl.ds(r0, ROWS_PER_TILE)])

    return agg


_sc_agg_hid = _make_sc_agg(D_HID)
_sc_agg_out = _make_sc_agg(D_PAD)


def _tc1_body(x_ref, w_ref, cnt_ref, g_ref, dinv_ref):
    i = pl.program_id(0)
    h = jnp.dot(x_ref[...], w_ref[...], preferred_element_type=jnp.float32)
    csum = cnt_ref[:, 0:1] + cnt_ref[:, 1:2]
    rows = jax.lax.broadcasted_iota(jnp.int32, (BLK_R, 1), 0) + i * BLK_R
    dinv = jnp.where(rows < N, jax.lax.rsqrt(csum + 1.0), 0.0)
    dinv_ref[...] = dinv
    g_ref[...] = h * dinv


_tc1 = pl.pallas_call(
    _tc1_body,
    grid=(NPAD // BLK_R,),
    in_specs=[
        pl.BlockSpec((BLK_R, D_IN), lambda i: (i, 0)),
        pl.BlockSpec((D_IN, D_HID), lambda i: (0, 0)),
        pl.BlockSpec((BLK_R, 2), lambda i: (i, 0)),
    ],
    out_specs=[
        pl.BlockSpec((BLK_R, D_HID), lambda i: (i, 0)),
        pl.BlockSpec((BLK_R, 1), lambda i: (i, 0)),
    ],
    out_shape=[
        jax.ShapeDtypeStruct((NPAD, D_HID), jnp.float32),
        jax.ShapeDtypeStruct((NPAD, 1), jnp.float32),
    ],
)


def _tc2_body(a0_ref, a1_ref, g1_ref, dinv_ref, b1_ref, w2_ref, g2_ref):
    agg = a0_ref[...] + a1_ref[...] + g1_ref[...]
    dinv = dinv_ref[...]
    h1 = jnp.maximum(agg * dinv + b1_ref[...], 0.0)
    g2_ref[...] = jnp.dot(h1, w2_ref[...],
                          preferred_element_type=jnp.float32) * dinv


_tc2 = pl.pallas_call(
    _tc2_body,
    grid=(NPAD // BLK_R,),
    in_specs=[
        pl.BlockSpec((BLK_R, D_HID), lambda i: (i, 0)),
        pl.BlockSpec((BLK_R, D_HID), lambda i: (i, 0)),
        pl.BlockSpec((BLK_R, D_HID), lambda i: (i, 0)),
        pl.BlockSpec((BLK_R, 1), lambda i: (i, 0)),
        pl.BlockSpec((1, D_HID), lambda i: (0, 0)),
        pl.BlockSpec((D_HID, D_PAD), lambda i: (0, 0)),
    ],
    out_specs=pl.BlockSpec((BLK_R, D_PAD), lambda i: (i, 0)),
    out_shape=jax.ShapeDtypeStruct((NPAD, D_PAD), jnp.float32),
)


def _tc3_body(a0_ref, a1_ref, g2_ref, dinv_ref, b2_ref, o_ref):
    o = (a0_ref[...] + a1_ref[...] + g2_ref[...]) * dinv_ref[...] + b2_ref[...]
    m = jnp.max(o, axis=1, keepdims=True)
    lse = m + jnp.log(jnp.sum(jnp.exp(o - m), axis=1, keepdims=True))
    o_ref[...] = o - lse


_tc3 = pl.pallas_call(
    _tc3_body,
    grid=(NPAD // BLK_R,),
    in_specs=[
        pl.BlockSpec((BLK_R, D_PAD), lambda i: (i, 0)),
        pl.BlockSpec((BLK_R, D_PAD), lambda i: (i, 0)),
        pl.BlockSpec((BLK_R, D_PAD), lambda i: (i, 0)),
        pl.BlockSpec((BLK_R, 1), lambda i: (i, 0)),
        pl.BlockSpec((1, D_PAD), lambda i: (0, 0)),
    ],
    out_specs=pl.BlockSpec((BLK_R, D_PAD), lambda i: (i, 0)),
    out_shape=jax.ShapeDtypeStruct((NPAD, D_PAD), jnp.float32),
)


def kernel(x, edge_index, W1, b1, W2, b2):
    pad_idx = jnp.full((EPAD - E,), NPAD - 1, dtype=jnp.int32)
    src2 = jnp.concatenate([edge_index[0], pad_idx]).reshape(EROWS, CH)
    dst2 = jnp.concatenate([edge_index[1], pad_idx]).reshape(EROWS, CH)

    cnt = _sc_degree(dst2, jnp.zeros((NPAD,), jnp.float32))
    xpad = jnp.concatenate(
        [x, jnp.zeros((NPAD - N, D_IN), jnp.float32)], axis=0)
    g1, dinv = _tc1(xpad, W1, cnt.T)

    agg1 = _sc_agg_hid(g1, src2, dst2, jnp.zeros((NPAD, D_HID), jnp.float32))
    g2 = _tc2(agg1[0], agg1[1], g1, dinv, b1.reshape(1, D_HID), W2)

    agg2 = _sc_agg_out(g2, src2, dst2, jnp.zeros((NPAD, D_PAD), jnp.float32))
    o = _tc3(agg2[0], agg2[1], g2, dinv, b2.reshape(1, D_PAD))
    return o[:N]


# speedup vs baseline: 39.3675x; 1.5957x over previous
import functools

import jax
import jax.numpy as jnp
from jax import lax
from jax.experimental import pallas as pl
from jax.experimental.pallas import tpu as pltpu
from jax.experimental.pallas import tpu_sc as plsc

N = 10000
E = 320000
D_IN = 128
D_HID = 16
D_OUT = 40

NC, NS = 2, 16
CH = 128
IDX_BLK = 8
NPAD = 10240
ROWS_PER_TILE = NPAD // NS
EROWS = E // CH

SLOW_CORE = 0
K_SLOW = 40
K_FAST = 116
FAST_BASE = NS * K_SLOW
EXTRA_BASE = FAST_BASE + NS * K_FAST

BLK_R = 1000


_SC_MESH = plsc.VectorSubcoreMesh(core_axis_name="c", subcore_axis_name="s")
_SC_PARAMS = pltpu.CompilerParams(use_tc_tiling_on_sc=False)


@functools.partial(
    pl.kernel,
    out_type=jax.ShapeDtypeStruct((NC, NPAD), jnp.float32),
    mesh=_SC_MESH,
    compiler_params=_SC_PARAMS,
    scratch_types=[
        pltpu.VMEM((IDX_BLK, CH), jnp.int32),
        pltpu.VMEM((CH,), jnp.float32),
        pltpu.VMEM_SHARED((NPAD,), jnp.float32),
    ],
)
def _sc_degree(e3_hbm, zero_hbm, out_hbm, didx_v, ones_v, acc_sh):
    c = lax.axis_index("c")
    s = lax.axis_index("s")
    w = c * NS + s
    r0 = s * ROWS_PER_TILE
    for i in range(CH // 16):
        ones_v[pl.ds(i * 16, 16)] = jnp.ones((16,), jnp.float32)
    pltpu.sync_copy(zero_hbm.at[pl.ds(r0, ROWS_PER_TILE)],
                    acc_sh.at[pl.ds(r0, ROWS_PER_TILE)])
    plsc.subcore_barrier()

    base = w * 78

    def blk(i, carry):
        row0 = base + i * IDX_BLK
        pltpu.sync_copy(e3_hbm.at[1, pl.ds(row0, IDX_BLK)], didx_v)
        for j in range(IDX_BLK):
            pltpu.sync_copy(ones_v, acc_sh.at[didx_v.at[j]], add=True)
        return carry

    lax.fori_loop(0, 9, blk, 0)
    pltpu.sync_copy(e3_hbm.at[1, pl.ds(base + 72, 6)],
                    didx_v.at[pl.ds(0, 6)])
    for j in range(6):
        pltpu.sync_copy(ones_v, acc_sh.at[didx_v.at[j]], add=True)

    @pl.when(w < 4)
    def _extra():
        pltpu.sync_copy(e3_hbm.at[1, pl.ds(2496 + w, 1)],
                        didx_v.at[pl.ds(0, 1)])
        pltpu.sync_copy(ones_v, acc_sh.at[didx_v.at[0]], add=True)

    plsc.subcore_barrier()
    pltpu.sync_copy(acc_sh.at[pl.ds(r0, ROWS_PER_TILE)],
                    out_hbm.at[c, pl.ds(r0, ROWS_PER_TILE)])


def _make_sc_agg(depth):
    @functools.partial(
        pl.kernel,
        out_type=jax.ShapeDtypeStruct((NC, NPAD, depth), jnp.float32),
        mesh=_SC_MESH,
        compiler_params=_SC_PARAMS,
        scratch_types=(
            [pltpu.VMEM((IDX_BLK, CH), jnp.int32),
             pltpu.VMEM((IDX_BLK, CH), jnp.int32)]
            + [pltpu.VMEM((CH, depth), jnp.float32) for _ in range(IDX_BLK)]
            + [pltpu.VMEM_SHARED((NPAD, depth), jnp.float32)]
            + [pltpu.SemaphoreType.DMA for _ in range(IDX_BLK + 2)]
        ),
    )
    def agg(g_hbm, e3_hbm, zero_hbm, out_hbm, sidx_v, didx_v, *scr):
        rows = scr[:IDX_BLK]
        acc_sh = scr[IDX_BLK]
        gsem = scr[IDX_BLK + 1:2 * IDX_BLK + 1]
        ssem = scr[2 * IDX_BLK + 1:]
        c = lax.axis_index("c")
        s = lax.axis_index("s")
        r0 = s * ROWS_PER_TILE
        grp = IDX_BLK // 2
        pltpu.sync_copy(zero_hbm.at[pl.ds(r0, ROWS_PER_TILE)],
                        acc_sh.at[pl.ds(r0, ROWS_PER_TILE)])
        plsc.subcore_barrier()

        def drain_scat(g):
            for _ in range(grp):
                pltpu.make_async_copy(
                    g_hbm.at[pl.ds(0, CH)], rows[0], ssem[g]).wait()

        def pipelined(base, nblk):
            def blk(i, carry):
                row0 = base + i * IDX_BLK
                pltpu.sync_copy(e3_hbm.at[0, pl.ds(row0, IDX_BLK)], sidx_v)
                pltpu.sync_copy(e3_hbm.at[1, pl.ds(row0, IDX_BLK)], didx_v)
                for g in range(2):
                    @pl.when(i >= 1)
                    def _(g=g):
                        drain_scat(g)
                    cps = []
                    for j in range(grp):
                        k = g * grp + j
                        cps.append(pltpu.async_copy(
                            g_hbm.at[sidx_v.at[k]], rows[k], gsem[k]))
                    for j in range(grp):
                        k = g * grp + j
                        cps[j].wait()
                        pltpu.async_copy(rows[k], acc_sh.at[didx_v.at[k]],
                                         ssem[g], add=True)
                return carry

            lax.fori_loop(0, nblk, blk, 0)
            for g in range(2):
                drain_scat(g)

        def tail(row0, count):
            pltpu.sync_copy(e3_hbm.at[0, pl.ds(row0, count)],
                            sidx_v.at[pl.ds(0, count)])
            pltpu.sync_copy(e3_hbm.at[1, pl.ds(row0, count)],
                            didx_v.at[pl.ds(0, count)])
            for j in range(count):
                pltpu.sync_copy(g_hbm.at[sidx_v.at[j]], rows[0])
                pltpu.sync_copy(rows[0], acc_sh.at[didx_v.at[j]], add=True)

        @pl.when(c == SLOW_CORE)
        def _slow():
            pipelined(s * K_SLOW, K_SLOW // IDX_BLK)

        @pl.when(c != SLOW_CORE)
        def _fast():
            pipelined(FAST_BASE + s * K_FAST, K_FAST // IDX_BLK)
            tail(FAST_BASE + s * K_FAST + (K_FAST // IDX_BLK) * IDX_BLK,
                 K_FAST % IDX_BLK)

            @pl.when(s < 4)
            def _extra():
                tail(EXTRA_BASE + s, 1)

        plsc.subcore_barrier()
        pltpu.sync_copy(acc_sh.at[pl.ds(r0, ROWS_PER_TILE)],
                        out_hbm.at[c, pl.ds(r0, ROWS_PER_TILE)])

    return agg


_sc_agg_hid = _make_sc_agg(D_HID)
_sc_agg_out = _make_sc_agg(D_OUT)


def _tc1_body(x_ref, w_ref, cnt_ref, g_ref, dinv_ref):
    h = jnp.dot(x_ref[...], w_ref[...], preferred_element_type=jnp.float32)
    csum = cnt_ref[:, 0:1] + cnt_ref[:, 1:2]
    dinv = jax.lax.rsqrt(csum + 1.0)
    dinv_ref[...] = dinv
    g_ref[...] = h * dinv


_tc1 = pl.pallas_call(
    _tc1_body,
    grid=(N // BLK_R,),
    in_specs=[
        pl.BlockSpec((BLK_R, D_IN), lambda i: (i, 0)),
        pl.BlockSpec((D_IN, D_HID), lambda i: (0, 0)),
        pl.BlockSpec((BLK_R, 2), lambda i: (i, 0)),
    ],
    out_specs=[
        pl.BlockSpec((BLK_R, D_HID), lambda i: (i, 0)),
        pl.BlockSpec((BLK_R, 1), lambda i: (i, 0)),
    ],
    out_shape=[
        jax.ShapeDtypeStruct((N, D_HID), jnp.float32),
        jax.ShapeDtypeStruct((N, 1), jnp.float32),
    ],
)


def _tc2_body(agg_ref0, agg_ref1, g1_ref, dinv_ref, b1_ref, w2_ref, g2_ref):
    agg = agg_ref0[0] + agg_ref1[0] + g1_ref[...]
    dinv = dinv_ref[...]
    h1 = jnp.maximum(agg * dinv + b1_ref[...], 0.0)
    g2_ref[...] = jnp.dot(h1, w2_ref[...],
                          preferred_element_type=jnp.float32) * dinv


_tc2 = pl.pallas_call(
    _tc2_body,
    grid=(N // BLK_R,),
    in_specs=[
        pl.BlockSpec((1, BLK_R, D_HID), lambda i: (0, i, 0)),
        pl.BlockSpec((1, BLK_R, D_HID), lambda i: (1, i, 0)),
        pl.BlockSpec((BLK_R, D_HID), lambda i: (i, 0)),
        pl.BlockSpec((BLK_R, 1), lambda i: (i, 0)),
        pl.BlockSpec((1, D_HID), lambda i: (0, 0)),
        pl.BlockSpec((D_HID, D_OUT), lambda i: (0, 0)),
    ],
    out_specs=pl.BlockSpec((BLK_R, D_OUT), lambda i: (i, 0)),
    out_shape=jax.ShapeDtypeStruct((N, D_OUT), jnp.float32),
)


def _tc3_body(agg_ref0, agg_ref1, g2_ref, dinv_ref, b2_ref, o_ref):
    o = (agg_ref0[0] + agg_ref1[0] + g2_ref[...]) * dinv_ref[...] + b2_ref[...]
    m = jnp.max(o, axis=1, keepdims=True)
    lse = m + jnp.log(jnp.sum(jnp.exp(o - m), axis=1, keepdims=True))
    o_ref[...] = o - lse


_tc3 = pl.pallas_call(
    _tc3_body,
    grid=(N // BLK_R,),
    in_specs=[
        pl.BlockSpec((1, BLK_R, D_OUT), lambda i: (0, i, 0)),
        pl.BlockSpec((1, BLK_R, D_OUT), lambda i: (1, i, 0)),
        pl.BlockSpec((BLK_R, D_OUT), lambda i: (i, 0)),
        pl.BlockSpec((BLK_R, 1), lambda i: (i, 0)),
        pl.BlockSpec((1, D_OUT), lambda i: (0, 0)),
    ],
    out_specs=pl.BlockSpec((BLK_R, D_OUT), lambda i: (i, 0)),
    out_shape=jax.ShapeDtypeStruct((N, D_OUT), jnp.float32),
)


def kernel(x, edge_index, W1, b1, W2, b2):
    e3 = edge_index.reshape(2, EROWS, CH)

    cnt = _sc_degree(e3, jnp.zeros((NPAD,), jnp.float32))
    g1, dinv = _tc1(x, W1, cnt.T[:N])

    agg1 = _sc_agg_hid(g1, e3, jnp.zeros((NPAD, D_HID), jnp.float32))
    g2 = _tc2(agg1, agg1, g1, dinv, b1.reshape(1, D_HID), W2)

    agg2 = _sc_agg_out(g2, e3, jnp.zeros((NPAD, D_OUT), jnp.float32))
    return _tc3(agg2, agg2, g2, dinv, b2.reshape(1, D_OUT))


# speedup vs baseline: 44.7368x; 1.1364x over previous
import functools

import jax
import jax.numpy as jnp
from jax import lax
from jax.experimental import pallas as pl
from jax.experimental.pallas import tpu as pltpu
from jax.experimental.pallas import tpu_sc as plsc

N = 10000
E = 320000
D_IN = 128
D_HID = 16
D_OUT = 40

NC, NS = 2, 16
CH = 128
IDX_BLK = 8
NPAD = 10240
ROWS_PER_TILE = NPAD // NS
EROWS = E // CH

BLK_R = 1000


_SC_MESH = plsc.VectorSubcoreMesh(core_axis_name="c", subcore_axis_name="s")
_SC_PARAMS = pltpu.CompilerParams(use_tc_tiling_on_sc=False)


@functools.partial(
    pl.kernel,
    out_type=jax.ShapeDtypeStruct((NC, NPAD), jnp.float32),
    mesh=_SC_MESH,
    compiler_params=_SC_PARAMS,
    scratch_types=[
        pltpu.VMEM((IDX_BLK, CH), jnp.int32),
        pltpu.VMEM((CH,), jnp.float32),
        pltpu.VMEM_SHARED((NPAD,), jnp.float32),
    ],
)
def _sc_degree(e3_hbm, zero_hbm, out_hbm, didx_v, ones_v, acc_sh):
    c = lax.axis_index("c")
    s = lax.axis_index("s")
    w = c * NS + s
    r0 = s * ROWS_PER_TILE
    for i in range(CH // 16):
        ones_v[pl.ds(i * 16, 16)] = jnp.ones((16,), jnp.float32)
    pltpu.sync_copy(zero_hbm.at[pl.ds(r0, ROWS_PER_TILE)],
                    acc_sh.at[pl.ds(r0, ROWS_PER_TILE)])
    plsc.subcore_barrier()

    base = w * 78

    def blk(i, carry):
        row0 = base + i * IDX_BLK
        pltpu.sync_copy(e3_hbm.at[1, pl.ds(row0, IDX_BLK)], didx_v)
        for j in range(IDX_BLK):
            pltpu.sync_copy(ones_v, acc_sh.at[didx_v.at[j]], add=True)
        return carry

    lax.fori_loop(0, 9, blk, 0)
    pltpu.sync_copy(e3_hbm.at[1, pl.ds(base + 72, 6)],
                    didx_v.at[pl.ds(0, 6)])
    for j in range(6):
        pltpu.sync_copy(ones_v, acc_sh.at[didx_v.at[j]], add=True)

    @pl.when(w < 4)
    def _extra():
        pltpu.sync_copy(e3_hbm.at[1, pl.ds(2496 + w, 1)],
                        didx_v.at[pl.ds(0, 1)])
        pltpu.sync_copy(ones_v, acc_sh.at[didx_v.at[0]], add=True)

    plsc.subcore_barrier()
    pltpu.sync_copy(acc_sh.at[pl.ds(r0, ROWS_PER_TILE)],
                    out_hbm.at[c, pl.ds(r0, ROWS_PER_TILE)])


def _make_sc_agg(depth):
    @functools.partial(
        pl.kernel,
        out_type=jax.ShapeDtypeStruct((NC, NPAD, depth), jnp.float32),
        mesh=_SC_MESH,
        compiler_params=_SC_PARAMS,
        scratch_types=(
            [pltpu.VMEM((IDX_BLK, CH), jnp.int32),
             pltpu.VMEM((IDX_BLK, CH), jnp.int32)]
            + [pltpu.VMEM((CH, depth), jnp.float32) for _ in range(IDX_BLK)]
            + [pltpu.VMEM_SHARED((NPAD, depth), jnp.float32)]
            + [pltpu.SemaphoreType.DMA for _ in range(IDX_BLK + 2)]
        ),
    )
    def agg(g_hbm, e3_hbm, zero_hbm, out_hbm, sidx_v, didx_v, *scr):
        rows = scr[:IDX_BLK]
        acc_sh = scr[IDX_BLK]
        gsem = scr[IDX_BLK + 1:2 * IDX_BLK + 1]
        ssem = scr[2 * IDX_BLK + 1:]
        c = lax.axis_index("c")
        s = lax.axis_index("s")
        r0 = s * ROWS_PER_TILE
        grp = IDX_BLK // 2
        pltpu.sync_copy(zero_hbm.at[pl.ds(r0, ROWS_PER_TILE)],
                        acc_sh.at[pl.ds(r0, ROWS_PER_TILE)])
        plsc.subcore_barrier()

        def drain_scat(g):
            for _ in range(grp):
                pltpu.make_async_copy(
                    g_hbm.at[pl.ds(0, CH)], rows[0], ssem[g]).wait()

        def pipelined(base, nblk):
            def blk(i, carry):
                row0 = base + i * IDX_BLK
                pltpu.sync_copy(e3_hbm.at[0, pl.ds(row0, IDX_BLK)], sidx_v)
                pltpu.sync_copy(e3_hbm.at[1, pl.ds(row0, IDX_BLK)], didx_v)
                for g in range(2):
                    @pl.when(i >= 1)
                    def _(g=g):
                        drain_scat(g)
                    cps = []
                    for j in range(grp):
                        k = g * grp + j
                        cps.append(pltpu.async_copy(
                            g_hbm.at[sidx_v.at[k]], rows[k], gsem[k]))
                    for j in range(grp):
                        k = g * grp + j
                        cps[j].wait()
                        pltpu.async_copy(rows[k], acc_sh.at[didx_v.at[k]],
                                         ssem[g], add=True)
                return carry

            lax.fori_loop(0, nblk, blk, 0)
            for g in range(2):
                drain_scat(g)

        def tail(row0, count):
            pltpu.sync_copy(e3_hbm.at[0, pl.ds(row0, count)],
                            sidx_v.at[pl.ds(0, count)])
            pltpu.sync_copy(e3_hbm.at[1, pl.ds(row0, count)],
                            didx_v.at[pl.ds(0, count)])
            for j in range(count):
                pltpu.sync_copy(g_hbm.at[sidx_v.at[j]], rows[0])
                pltpu.sync_copy(rows[0], acc_sh.at[didx_v.at[j]], add=True)

        w = c * NS + s
        pipelined(w * 78, 9)
        tail(w * 78 + 72, 6)

        @pl.when(w < 4)
        def _extra():
            tail(2496 + w, 1)

        plsc.subcore_barrier()
        pltpu.sync_copy(acc_sh.at[pl.ds(r0, ROWS_PER_TILE)],
                        out_hbm.at[c, pl.ds(r0, ROWS_PER_TILE)])

    return agg


_sc_agg_hid = _make_sc_agg(D_HID)
_sc_agg_out = _make_sc_agg(D_OUT)


def _tc1_body(x_ref, w_ref, cnt_ref, g_ref, dinv_ref):
    h = jnp.dot(x_ref[...], w_ref[...], preferred_element_type=jnp.float32)
    csum = cnt_ref[:, 0:1] + cnt_ref[:, 1:2]
    dinv = jax.lax.rsqrt(csum + 1.0)
    dinv_ref[...] = dinv
    g_ref[...] = h * dinv


_tc1 = pl.pallas_call(
    _tc1_body,
    grid=(N // BLK_R,),
    in_specs=[
        pl.BlockSpec((BLK_R, D_IN), lambda i: (i, 0)),
        pl.BlockSpec((D_IN, D_HID), lambda i: (0, 0)),
        pl.BlockSpec((BLK_R, 2), lambda i: (i, 0)),
    ],
    out_specs=[
        pl.BlockSpec((BLK_R, D_HID), lambda i: (i, 0)),
        pl.BlockSpec((BLK_R, 1), lambda i: (i, 0)),
    ],
    out_shape=[
        jax.ShapeDtypeStruct((N, D_HID), jnp.float32),
        jax.ShapeDtypeStruct((N, 1), jnp.float32),
    ],
)


def _tc2_body(agg_ref0, agg_ref1, g1_ref, dinv_ref, b1_ref, w2_ref, g2_ref):
    agg = agg_ref0[0] + agg_ref1[0] + g1_ref[...]
    dinv = dinv_ref[...]
    h1 = jnp.maximum(agg * dinv + b1_ref[...], 0.0)
    g2_ref[...] = jnp.dot(h1, w2_ref[...],
                          preferred_element_type=jnp.float32) * dinv


_tc2 = pl.pallas_call(
    _tc2_body,
    grid=(N // BLK_R,),
    in_specs=[
        pl.BlockSpec((1, BLK_R, D_HID), lambda i: (0, i, 0)),
        pl.BlockSpec((1, BLK_R, D_HID), lambda i: (1, i, 0)),
        pl.BlockSpec((BLK_R, D_HID), lambda i: (i, 0)),
        pl.BlockSpec((BLK_R, 1), lambda i: (i, 0)),
        pl.BlockSpec((1, D_HID), lambda i: (0, 0)),
        pl.BlockSpec((D_HID, D_OUT), lambda i: (0, 0)),
    ],
    out_specs=pl.BlockSpec((BLK_R, D_OUT), lambda i: (i, 0)),
    out_shape=jax.ShapeDtypeStruct((N, D_OUT), jnp.float32),
)


def _tc3_body(agg_ref0, agg_ref1, g2_ref, dinv_ref, b2_ref, o_ref):
    o = (agg_ref0[0] + agg_ref1[0] + g2_ref[...]) * dinv_ref[...] + b2_ref[...]
    m = jnp.max(o, axis=1, keepdims=True)
    lse = m + jnp.log(jnp.sum(jnp.exp(o - m), axis=1, keepdims=True))
    o_ref[...] = o - lse


_tc3 = pl.pallas_call(
    _tc3_body,
    grid=(N // BLK_R,),
    in_specs=[
        pl.BlockSpec((1, BLK_R, D_OUT), lambda i: (0, i, 0)),
        pl.BlockSpec((1, BLK_R, D_OUT), lambda i: (1, i, 0)),
        pl.BlockSpec((BLK_R, D_OUT), lambda i: (i, 0)),
        pl.BlockSpec((BLK_R, 1), lambda i: (i, 0)),
        pl.BlockSpec((1, D_OUT), lambda i: (0, 0)),
    ],
    out_specs=pl.BlockSpec((BLK_R, D_OUT), lambda i: (i, 0)),
    out_shape=jax.ShapeDtypeStruct((N, D_OUT), jnp.float32),
)


def kernel(x, edge_index, W1, b1, W2, b2):
    e3 = edge_index.reshape(2, EROWS, CH)

    cnt = _sc_degree(e3, jnp.zeros((NPAD,), jnp.float32))
    g1, dinv = _tc1(x, W1, cnt.T[:N])

    agg1 = _sc_agg_hid(g1, e3, jnp.zeros((NPAD, D_HID), jnp.float32))
    g2 = _tc2(agg1, agg1, g1, dinv, b1.reshape(1, D_HID), W2)

    agg2 = _sc_agg_out(g2, e3, jnp.zeros((NPAD, D_OUT), jnp.float32))
    return _tc3(agg2, agg2, g2, dinv, b2.reshape(1, D_OUT))


# speedup vs baseline: 45.7612x; 1.0229x over previous
import functools

import jax
import jax.numpy as jnp
from jax import lax
from jax.experimental import pallas as pl
from jax.experimental.pallas import tpu as pltpu
from jax.experimental.pallas import tpu_sc as plsc

N = 10000
E = 320000
D_IN = 128
D_HID = 16
D_OUT = 40

NC, NS = 2, 16
CH = 128
IDX_BLK = 8
NPAD = 10240
ROWS_PER_TILE = NPAD // NS
EROWS = E // CH


_SC_MESH = plsc.VectorSubcoreMesh(core_axis_name="c", subcore_axis_name="s")
_SC_PARAMS = pltpu.CompilerParams(use_tc_tiling_on_sc=False)


@functools.partial(
    pl.kernel,
    out_type=jax.ShapeDtypeStruct((NC, NPAD), jnp.float32),
    mesh=_SC_MESH,
    compiler_params=_SC_PARAMS,
    scratch_types=[
        pltpu.VMEM((IDX_BLK, CH), jnp.int32),
        pltpu.VMEM((CH,), jnp.float32),
        pltpu.VMEM_SHARED((NPAD,), jnp.float32),
    ],
)
def _sc_degree(e3_hbm, zero_hbm, out_hbm, didx_v, ones_v, acc_sh):
    c = lax.axis_index("c")
    s = lax.axis_index("s")
    w = c * NS + s
    r0 = s * ROWS_PER_TILE
    for i in range(CH // 16):
        ones_v[pl.ds(i * 16, 16)] = jnp.ones((16,), jnp.float32)
    pltpu.sync_copy(zero_hbm.at[pl.ds(r0, ROWS_PER_TILE)],
                    acc_sh.at[pl.ds(r0, ROWS_PER_TILE)])
    plsc.subcore_barrier()

    base = w * 78

    def blk(i, carry):
        row0 = base + i * IDX_BLK
        pltpu.sync_copy(e3_hbm.at[1, pl.ds(row0, IDX_BLK)], didx_v)
        for j in range(IDX_BLK):
            pltpu.sync_copy(ones_v, acc_sh.at[didx_v.at[j]], add=True)
        return carry

    lax.fori_loop(0, 9, blk, 0)
    pltpu.sync_copy(e3_hbm.at[1, pl.ds(base + 72, 6)],
                    didx_v.at[pl.ds(0, 6)])
    for j in range(6):
        pltpu.sync_copy(ones_v, acc_sh.at[didx_v.at[j]], add=True)

    @pl.when(w < 4)
    def _extra():
        pltpu.sync_copy(e3_hbm.at[1, pl.ds(2496 + w, 1)],
                        didx_v.at[pl.ds(0, 1)])
        pltpu.sync_copy(ones_v, acc_sh.at[didx_v.at[0]], add=True)

    plsc.subcore_barrier()
    pltpu.sync_copy(acc_sh.at[pl.ds(r0, ROWS_PER_TILE)],
                    out_hbm.at[c, pl.ds(r0, ROWS_PER_TILE)])


def _make_sc_agg(depth):
    @functools.partial(
        pl.kernel,
        out_type=jax.ShapeDtypeStruct((NC, NPAD, depth), jnp.float32),
        mesh=_SC_MESH,
        compiler_params=_SC_PARAMS,
        scratch_types=(
            [pltpu.VMEM((IDX_BLK, CH), jnp.int32),
             pltpu.VMEM((IDX_BLK, CH), jnp.int32)]
            + [pltpu.VMEM((CH, depth), jnp.float32) for _ in range(IDX_BLK)]
            + [pltpu.VMEM_SHARED((NPAD, depth), jnp.float32)]
            + [pltpu.SemaphoreType.DMA for _ in range(IDX_BLK + 2)]
        ),
    )
    def agg(g_hbm, e3_hbm, zero_hbm, out_hbm, sidx_v, didx_v, *scr):
        rows = scr[:IDX_BLK]
        acc_sh = scr[IDX_BLK]
        gsem = scr[IDX_BLK + 1:2 * IDX_BLK + 1]
        ssem = scr[2 * IDX_BLK + 1:]
        c = lax.axis_index("c")
        s = lax.axis_index("s")
        r0 = s * ROWS_PER_TILE
        grp = IDX_BLK // 2
        pltpu.sync_copy(zero_hbm.at[pl.ds(r0, ROWS_PER_TILE)],
                        acc_sh.at[pl.ds(r0, ROWS_PER_TILE)])
        plsc.subcore_barrier()

        def drain_scat(g):
            for _ in range(grp):
                pltpu.make_async_copy(
                    g_hbm.at[pl.ds(0, CH)], rows[0], ssem[g]).wait()

        def pipelined(base, nblk):
            def blk(i, carry):
                row0 = base + i * IDX_BLK
                pltpu.sync_copy(e3_hbm.at[0, pl.ds(row0, IDX_BLK)], sidx_v)
                pltpu.sync_copy(e3_hbm.at[1, pl.ds(row0, IDX_BLK)], didx_v)
                for g in range(2):
                    @pl.when(i >= 1)
                    def _(g=g):
                        drain_scat(g)
                    cps = []
                    for j in range(grp):
                        k = g * grp + j
                        cps.append(pltpu.async_copy(
                            g_hbm.at[sidx_v.at[k]], rows[k], gsem[k]))
                    for j in range(grp):
                        k = g * grp + j
                        cps[j].wait()
                        pltpu.async_copy(rows[k], acc_sh.at[didx_v.at[k]],
                                         ssem[g], add=True)
                return carry

            lax.fori_loop(0, nblk, blk, 0)
            for g in range(2):
                drain_scat(g)

        def tail(row0, count):
            pltpu.sync_copy(e3_hbm.at[0, pl.ds(row0, count)],
                            sidx_v.at[pl.ds(0, count)])
            pltpu.sync_copy(e3_hbm.at[1, pl.ds(row0, count)],
                            didx_v.at[pl.ds(0, count)])
            for j in range(count):
                pltpu.sync_copy(g_hbm.at[sidx_v.at[j]], rows[0])
                pltpu.sync_copy(rows[0], acc_sh.at[didx_v.at[j]], add=True)

        w = c * NS + s
        pipelined(w * 78, 9)
        tail(w * 78 + 72, 6)

        @pl.when(w < 4)
        def _extra():
            tail(2496 + w, 1)

        plsc.subcore_barrier()
        pltpu.sync_copy(acc_sh.at[pl.ds(r0, ROWS_PER_TILE)],
                        out_hbm.at[c, pl.ds(r0, ROWS_PER_TILE)])

    return agg


_sc_agg_hid = _make_sc_agg(D_HID)
_sc_agg_out = _make_sc_agg(D_OUT)


def _tc0_body(x_ref, w_ref, h_ref):
    h_ref[...] = jnp.dot(x_ref[...], w_ref[...],
                         preferred_element_type=jnp.float32)


_tc0 = pl.pallas_call(
    _tc0_body,
    out_shape=jax.ShapeDtypeStruct((N, D_HID), jnp.float32),
)


def _tc1_body(h_ref, cnt_ref, g_ref, dinv_ref):
    csum = cnt_ref[:, 0:1] + cnt_ref[:, 1:2]
    dinv = jax.lax.rsqrt(csum + 1.0)
    dinv_ref[...] = dinv
    g_ref[...] = h_ref[...] * dinv


_tc1 = pl.pallas_call(
    _tc1_body,
    out_shape=[
        jax.ShapeDtypeStruct((N, D_HID), jnp.float32),
        jax.ShapeDtypeStruct((N, 1), jnp.float32),
    ],
)


def _tc2_body(agg_ref, g1_ref, dinv_ref, b1_ref, w2_ref, g2_ref):
    agg = agg_ref[0, :N] + agg_ref[1, :N] + g1_ref[...]
    dinv = dinv_ref[...]
    h1 = jnp.maximum(agg * dinv + b1_ref[...], 0.0)
    g2_ref[...] = jnp.dot(h1, w2_ref[...],
                          preferred_element_type=jnp.float32) * dinv


_tc2 = pl.pallas_call(
    _tc2_body,
    out_shape=jax.ShapeDtypeStruct((N, D_OUT), jnp.float32),
)


def _tc3_body(agg_ref, g2_ref, dinv_ref, b2_ref, o_ref):
    o = (agg_ref[0, :N] + agg_ref[1, :N] + g2_ref[...]) * dinv_ref[...] \
        + b2_ref[...]
    m = jnp.max(o, axis=1, keepdims=True)
    lse = m + jnp.log(jnp.sum(jnp.exp(o - m), axis=1, keepdims=True))
    o_ref[...] = o - lse


_tc3 = pl.pallas_call(
    _tc3_body,
    out_shape=jax.ShapeDtypeStruct((N, D_OUT), jnp.float32),
)


def kernel(x, edge_index, W1, b1, W2, b2):
    e3 = edge_index.reshape(2, EROWS, CH)

    cnt = _sc_degree(e3, jnp.zeros((NPAD,), jnp.float32))
    h = _tc0(x, W1)
    g1, dinv = _tc1(h, cnt.T[:N])

    agg1 = _sc_agg_hid(g1, e3, jnp.zeros((NPAD, D_HID), jnp.float32))
    g2 = _tc2(agg1, g1, dinv, b1.reshape(1, D_HID), W2)

    agg2 = _sc_agg_out(g2, e3, jnp.zeros((NPAD, D_OUT), jnp.float32))
    return _tc3(agg2, g2, dinv, b2.reshape(1, D_OUT))
